# Initial kernel scaffold; baseline (speedup 1.0000x reference)
import jax
import jax.numpy as jnp
from jax.experimental import pallas as pl


def kernel(images, palettes, temperature):
    raise NotImplementedError("write your pallas kernel here")


# speedup vs baseline: 1.3887x; 1.3887x over previous
import jax
import jax.numpy as jnp
from jax.experimental import pallas as pl
from jax.experimental.pallas import tpu as pltpu

_K = 64
_ROWS = 128
_LANES = 128


def _tc_body(coef_ref, x_ref, o_ref):
    r = x_ref[0, 0]
    g = x_ref[0, 1]
    b = x_ref[0, 2]
    s = jnp.zeros_like(r)
    accr = jnp.zeros_like(r)
    accg = jnp.zeros_like(r)
    accb = jnp.zeros_like(r)
    for k in range(_K):
        ar = coef_ref[0, 0, k]
        ag = coef_ref[0, 1, k]
        ab = coef_ref[0, 2, k]
        b0 = coef_ref[0, 3, k]
        cr = coef_ref[0, 4, k]
        cg = coef_ref[0, 5, k]
        cb = coef_ref[0, 6, k]
        e = jnp.exp(r * ar + g * ag + b * ab + b0)
        s = s + e
        accr = accr + e * cr
        accg = accg + e * cg
        accb = accb + e * cb
    inv = 1.0 / s
    o_ref[0, 0] = accr * inv
    o_ref[0, 1] = accg * inv
    o_ref[0, 2] = accb * inv


def kernel(images, palettes, temperature):
    B, H, W, C = images.shape
    n = H * W
    rows = n // _LANES

    x = images.reshape(B, n, C).transpose(0, 2, 1).reshape(B, C, rows, _LANES)

    inv_t = 1.0 / temperature
    a = (2.0 * inv_t) * palettes
    b0 = -inv_t * jnp.sum(palettes * palettes, -1)
    coefs = jnp.concatenate(
        [a.transpose(0, 2, 1), b0[:, None, :], palettes.transpose(0, 2, 1),
         jnp.zeros((B, 1, _K), jnp.float32)], axis=1)

    grid = (B, rows // _ROWS)
    out = pl.pallas_call(
        _tc_body,
        grid=grid,
        in_specs=[
            pl.BlockSpec((1, 8, _K), lambda bi, i: (bi, 0, 0),
                         memory_space=pltpu.SMEM),
            pl.BlockSpec((1, C, _ROWS, _LANES), lambda bi, i: (bi, 0, i, 0)),
        ],
        out_specs=pl.BlockSpec((1, C, _ROWS, _LANES),
                               lambda bi, i: (bi, 0, i, 0)),
        out_shape=jax.ShapeDtypeStruct((B, C, rows, _LANES), jnp.float32),
    )(coefs, x)

    return out.reshape(B, C, n).transpose(0, 2, 1).reshape(B, H, W, C)
